# baseline (speedup 1.0000x reference)
import functools

import jax
import jax.numpy as jnp
import numpy as np
from jax import lax
from jax.experimental import pallas as pl
from jax.experimental.pallas import tpu as pltpu
from jax.experimental.pallas import tpu_sc as plsc

B = 4096
VOCAB = 100000
NNZ = 262144
UNITS = 64
D_DENSE = 128

NC = 2
NS = 16
NW = NC * NS
NNZ_W = NNZ // NW
CH = 128
NCH = NNZ_W // CH
NBUF = 4
HALF = NBUF // 2
ROWS_W = B // NS

_GATHER_DNUMS = lax.GatherDimensionNumbers(
    offset_dims=(), collapsed_slice_dims=(0,), start_index_map=(0,))


def _lane_splat(v16, lane):
    idx = jnp.full((16, 1), lane, jnp.int32)
    return lax.gather(
        v16, idx, _GATHER_DNUMS, slice_sizes=(1,),
        mode=lax.GatherScatterMode.PROMISE_IN_BOUNDS)


@functools.partial(
    pl.kernel,
    mesh=plsc.VectorSubcoreMesh(core_axis_name="c", subcore_axis_name="s"),
    compiler_params=pltpu.CompilerParams(use_tc_tiling_on_sc=False),
    out_type=jax.ShapeDtypeStruct((NC, B, 2 * UNITS), jnp.float32),
    scratch_types=[
        pltpu.VMEM((NNZ_W,), jnp.int32),
        pltpu.VMEM((NNZ_W,), jnp.int32),
        pltpu.VMEM((NNZ_W,), jnp.float32),
        [pltpu.VMEM((CH,), jnp.int32) for _ in range(NBUF)],
        [pltpu.VMEM((CH,), jnp.int32) for _ in range(NBUF)],
        [pltpu.VMEM((CH, UNITS), jnp.float32) for _ in range(NBUF)],
        pltpu.VMEM((16, UNITS), jnp.float32),
        pltpu.VMEM_SHARED((B, UNITS), jnp.float32),
        [pltpu.SemaphoreType.DMA for _ in range(NBUF)],
        [pltpu.SemaphoreType.DMA for _ in range(NBUF)],
    ],
)
def _sc_partial(cols_hbm, rows_hbm, vals_hbm, table_hbm, out_hbm,
                cols_all, rows_all, vals_all, gidx, sidx, gath, zbuf,
                acc, gsem, ssem):
    cid = lax.axis_index("c")
    sid = lax.axis_index("s")
    base = pl.multiple_of((cid * NS + sid) * NNZ_W, NNZ_W)

    pltpu.sync_copy(cols_hbm.at[pl.ds(base, NNZ_W)], cols_all)
    pltpu.sync_copy(rows_hbm.at[pl.ds(base, NNZ_W)], rows_all)
    pltpu.sync_copy(vals_hbm.at[pl.ds(base, NNZ_W)], vals_all)

    def _stage(c, b):
        o = c * CH
        if not isinstance(o, int):
            o = pl.multiple_of(o, CH)
        for t in range(CH // 16):
            gidx[b][pl.ds(t * 16, 16)] = cols_all[pl.ds(o + t * 16, 16)]
            sidx[b][pl.ds(t * 16, 16)] = rows_all[pl.ds(o + t * 16, 16)]

    for b in range(HALF):
        _stage(b, b)
        pltpu.async_copy(table_hbm.at[gidx[b]], gath[b], gsem[b])

    zero16 = jnp.zeros((16,), jnp.float32)
    for r in range(16):
        for q in range(UNITS // 16):
            zbuf[r, pl.ds(q * 16, 16)] = zero16
    row0 = sid * ROWS_W

    def _zero(t, carry):
        pltpu.sync_copy(zbuf, acc.at[pl.ds(row0 + t * 16, 16)])
        return carry

    lax.fori_loop(0, ROWS_W // 16, _zero, 0)
    plsc.subcore_barrier()

    def _pipe(i, carry):
        for b in range(NBUF):
            c = i * NBUF + b
            b2 = (b + HALF) % NBUF

            pltpu.make_async_copy(
                table_hbm.at[gidx[b]], gath[b], gsem[b]).wait()
            o = pl.multiple_of(c * CH, CH)

            @plsc.parallel_loop(0, CH // 16, unroll=4)
            def _scale(g, _b=b, _o=o):
                v16 = vals_all[pl.ds(_o + g * 16, 16)]
                for lane in range(16):
                    sp = _lane_splat(v16, lane)
                    j = g * 16 + lane
                    for q in range(UNITS // 16):
                        gath[_b][j, pl.ds(q * 16, 16)] = (
                            gath[_b][j, pl.ds(q * 16, 16)] * sp)
            pltpu.async_copy(gath[b], acc.at[sidx[b]], ssem[b], add=True)

            @pl.when(c < NCH - HALF)
            def _(c=c, b2=b2):
                @pl.when(c >= HALF)
                def _():
                    pltpu.make_async_copy(
                        gath[b2], acc.at[sidx[b2]], ssem[b2]).wait()
                _stage(c + HALF, b2)
                pltpu.async_copy(table_hbm.at[gidx[b2]], gath[b2], gsem[b2])
        return carry

    lax.fori_loop(0, NCH // NBUF, _pipe, 0)

    for b in range(NBUF):
        pltpu.make_async_copy(gath[b], acc.at[sidx[b]], ssem[b]).wait()

    plsc.subcore_barrier()
    pltpu.sync_copy(acc.at[pl.ds(row0, ROWS_W)],
                    out_hbm.at[cid, pl.ds(row0, ROWS_W), pl.ds(0, UNITS)])


def _concat_body(p_ref, d_ref, b_ref, o_ref):
    o_ref[:, :UNITS] = (p_ref[0, :, :UNITS] + p_ref[1, :, :UNITS]
                        + b_ref[...])
    o_ref[:, UNITS:] = d_ref[...]


_RB = 1024


def _concat(partials, dense_feat, bias2d):
    return pl.pallas_call(
        _concat_body,
        grid=(B // _RB,),
        in_specs=[
            pl.BlockSpec((NC, _RB, 2 * UNITS), lambda i: (0, i, 0)),
            pl.BlockSpec((_RB, D_DENSE), lambda i: (i, 0)),
            pl.BlockSpec((1, UNITS), lambda i: (0, 0)),
        ],
        out_specs=pl.BlockSpec((_RB, UNITS + D_DENSE), lambda i: (i, 0)),
        out_shape=jax.ShapeDtypeStruct((B, UNITS + D_DENSE), jnp.float32),
    )(partials, dense_feat, bias2d)


def kernel(sparse_rows, sparse_cols, sparse_values, dense_feat, kernel, bias):
    cols = sparse_cols.astype(jnp.int32)
    rows = sparse_rows.astype(jnp.int32)
    partials = _sc_partial(cols, rows, sparse_values, kernel)
    return _concat(partials, dense_feat, bias.reshape(1, UNITS))


# speedup vs baseline: 1.0077x; 1.0077x over previous
import functools

import jax
import jax.numpy as jnp
import numpy as np
from jax import lax
from jax.experimental import pallas as pl
from jax.experimental.pallas import tpu as pltpu
from jax.experimental.pallas import tpu_sc as plsc

B = 4096
VOCAB = 100000
NNZ = 262144
UNITS = 64
D_DENSE = 128

NC = 2
NS = 16
NW = NC * NS
NNZ_W = NNZ // NW
CH = 128
NCH = NNZ_W // CH
NBUF = 4
HALF = NBUF // 2
ROWS_W = B // NS

_GATHER_DNUMS = lax.GatherDimensionNumbers(
    offset_dims=(), collapsed_slice_dims=(0,), start_index_map=(0,))


def _lane_splat(v16, lane):
    idx = jnp.full((16, 1), lane, jnp.int32)
    return lax.gather(
        v16, idx, _GATHER_DNUMS, slice_sizes=(1,),
        mode=lax.GatherScatterMode.PROMISE_IN_BOUNDS)


@functools.partial(
    pl.kernel,
    mesh=plsc.VectorSubcoreMesh(core_axis_name="c", subcore_axis_name="s"),
    compiler_params=pltpu.CompilerParams(use_tc_tiling_on_sc=False),
    out_type=jax.ShapeDtypeStruct((NC, B, 2 * UNITS), jnp.float32),
    scratch_types=[
        pltpu.VMEM((NNZ_W,), jnp.int32),
        pltpu.VMEM((NNZ_W,), jnp.int32),
        pltpu.VMEM((NNZ_W,), jnp.float32),
        [pltpu.VMEM((CH,), jnp.int32) for _ in range(NBUF)],
        [pltpu.VMEM((CH,), jnp.int32) for _ in range(NBUF)],
        [pltpu.VMEM((CH, UNITS), jnp.float32) for _ in range(NBUF)],
        pltpu.VMEM((16, UNITS), jnp.float32),
        pltpu.VMEM_SHARED((B, UNITS), jnp.float32),
        [pltpu.SemaphoreType.DMA for _ in range(NBUF)],
        [pltpu.SemaphoreType.DMA for _ in range(NBUF)],
    ],
)
def _sc_partial(cols_hbm, rows_hbm, vals_hbm, table_hbm, out_hbm,
                cols_all, rows_all, vals_all, gidx, sidx, gath, zbuf,
                acc, gsem, ssem):
    cid = lax.axis_index("c")
    sid = lax.axis_index("s")
    base = pl.multiple_of((cid * NS + sid) * NNZ_W, NNZ_W)

    pltpu.sync_copy(cols_hbm.at[pl.ds(base, NNZ_W)], cols_all)
    pltpu.sync_copy(rows_hbm.at[pl.ds(base, NNZ_W)], rows_all)
    pltpu.sync_copy(vals_hbm.at[pl.ds(base, NNZ_W)], vals_all)

    def _stage(c, b):
        o = c * CH
        if not isinstance(o, int):
            o = pl.multiple_of(o, CH)
        for t in range(CH // 16):
            gidx[b][pl.ds(t * 16, 16)] = cols_all[pl.ds(o + t * 16, 16)]
            sidx[b][pl.ds(t * 16, 16)] = rows_all[pl.ds(o + t * 16, 16)]

    for b in range(HALF):
        _stage(b, b)
        pltpu.async_copy(table_hbm.at[gidx[b]], gath[b], gsem[b])

    zero16 = jnp.zeros((16,), jnp.float32)
    for r in range(16):
        for q in range(UNITS // 16):
            zbuf[r, pl.ds(q * 16, 16)] = zero16
    row0 = sid * ROWS_W

    def _zero(t, carry):
        pltpu.sync_copy(zbuf, acc.at[pl.ds(row0 + t * 16, 16)])
        return carry

    lax.fori_loop(0, ROWS_W // 16, _zero, 0)
    plsc.subcore_barrier()

    def _pipe(i, carry):
        for b in range(NBUF):
            c = i * NBUF + b
            b2 = (b + HALF) % NBUF

            pltpu.make_async_copy(
                table_hbm.at[gidx[b]], gath[b], gsem[b]).wait()
            o = pl.multiple_of(c * CH, CH)

            @plsc.parallel_loop(0, CH // 16, unroll=2)
            def _scale(g, _b=b, _o=o):
                v16 = vals_all[pl.ds(_o + g * 16, 16)]
                for lane in range(16):
                    sp = _lane_splat(v16, lane)
                    j = g * 16 + lane
                    for q in range(UNITS // 16):
                        gath[_b][j, pl.ds(q * 16, 16)] = (
                            gath[_b][j, pl.ds(q * 16, 16)] * sp)
            pltpu.async_copy(gath[b], acc.at[sidx[b]], ssem[b], add=True)

            @pl.when(c < NCH - HALF)
            def _(c=c, b2=b2):
                @pl.when(c >= HALF)
                def _():
                    pltpu.make_async_copy(
                        gath[b2], acc.at[sidx[b2]], ssem[b2]).wait()
                _stage(c + HALF, b2)
                pltpu.async_copy(table_hbm.at[gidx[b2]], gath[b2], gsem[b2])
        return carry

    lax.fori_loop(0, NCH // NBUF, _pipe, 0)

    for b in range(NBUF):
        pltpu.make_async_copy(gath[b], acc.at[sidx[b]], ssem[b]).wait()

    plsc.subcore_barrier()
    pltpu.sync_copy(acc.at[pl.ds(row0, ROWS_W)],
                    out_hbm.at[cid, pl.ds(row0, ROWS_W), pl.ds(0, UNITS)])


def _concat_body(p_ref, d_ref, b_ref, o_ref):
    o_ref[:, :UNITS] = (p_ref[0, :, :UNITS] + p_ref[1, :, :UNITS]
                        + b_ref[...])
    o_ref[:, UNITS:] = d_ref[...]


_RB = 1024


def _concat(partials, dense_feat, bias2d):
    return pl.pallas_call(
        _concat_body,
        grid=(B // _RB,),
        in_specs=[
            pl.BlockSpec((NC, _RB, 2 * UNITS), lambda i: (0, i, 0)),
            pl.BlockSpec((_RB, D_DENSE), lambda i: (i, 0)),
            pl.BlockSpec((1, UNITS), lambda i: (0, 0)),
        ],
        out_specs=pl.BlockSpec((_RB, UNITS + D_DENSE), lambda i: (i, 0)),
        out_shape=jax.ShapeDtypeStruct((B, UNITS + D_DENSE), jnp.float32),
    )(partials, dense_feat, bias2d)


def kernel(sparse_rows, sparse_cols, sparse_values, dense_feat, kernel, bias):
    cols = sparse_cols.astype(jnp.int32)
    rows = sparse_rows.astype(jnp.int32)
    partials = _sc_partial(cols, rows, sparse_values, kernel)
    return _concat(partials, dense_feat, bias.reshape(1, UNITS))


# speedup vs baseline: 1.0117x; 1.0040x over previous
import functools

import jax
import jax.numpy as jnp
from jax import lax
from jax.experimental import pallas as pl
from jax.experimental.pallas import tpu as pltpu
from jax.experimental.pallas import tpu_sc as plsc

B = 4096
VOCAB = 100000
NNZ = 262144
UNITS = 64
D_DENSE = 128

NC = 2
NS = 16
NW = NC * NS
NNZ_W = NNZ // NW
CH = 256
NCH = NNZ_W // CH
NBUF = 4
HALF = NBUF // 2
ROWS_W = B // NS

_GATHER_DNUMS = lax.GatherDimensionNumbers(
    offset_dims=(), collapsed_slice_dims=(0,), start_index_map=(0,))


def _lane_splat(v16, lane):
    idx = jnp.full((16, 1), lane, jnp.int32)
    return lax.gather(
        v16, idx, _GATHER_DNUMS, slice_sizes=(1,),
        mode=lax.GatherScatterMode.PROMISE_IN_BOUNDS)


@functools.partial(
    pl.kernel,
    mesh=plsc.VectorSubcoreMesh(core_axis_name="c", subcore_axis_name="s"),
    compiler_params=pltpu.CompilerParams(use_tc_tiling_on_sc=False),
    out_type=jax.ShapeDtypeStruct((NC, B, 2 * UNITS), jnp.float32),
    scratch_types=[
        pltpu.VMEM((NNZ_W,), jnp.int32),
        pltpu.VMEM((NNZ_W,), jnp.int32),
        pltpu.VMEM((NNZ_W,), jnp.float32),
        [pltpu.VMEM((CH,), jnp.int32) for _ in range(NBUF)],
        [pltpu.VMEM((CH,), jnp.int32) for _ in range(NBUF)],
        [pltpu.VMEM((CH, UNITS), jnp.float32) for _ in range(NBUF)],
        pltpu.VMEM((16, UNITS), jnp.float32),
        pltpu.VMEM_SHARED((B, UNITS), jnp.float32),
        [pltpu.SemaphoreType.DMA for _ in range(NBUF)],
        [pltpu.SemaphoreType.DMA for _ in range(NBUF)],
    ],
)
def _sc_partial(cols_hbm, rows_hbm, vals_hbm, table_hbm, out_hbm,
                cols_all, rows_all, vals_all, gidx, sidx, gath, zbuf,
                acc, gsem, ssem):
    cid = lax.axis_index("c")
    sid = lax.axis_index("s")
    base = pl.multiple_of((cid * NS + sid) * NNZ_W, NNZ_W)

    pltpu.sync_copy(cols_hbm.at[pl.ds(base, NNZ_W)], cols_all)
    pltpu.sync_copy(rows_hbm.at[pl.ds(base, NNZ_W)], rows_all)
    pltpu.sync_copy(vals_hbm.at[pl.ds(base, NNZ_W)], vals_all)

    def _stage(c, b):
        o = c * CH
        if not isinstance(o, int):
            o = pl.multiple_of(o, CH)
        for t in range(CH // 16):
            gidx[b][pl.ds(t * 16, 16)] = cols_all[pl.ds(o + t * 16, 16)]
            sidx[b][pl.ds(t * 16, 16)] = rows_all[pl.ds(o + t * 16, 16)]

    for b in range(HALF):
        _stage(b, b)
        pltpu.async_copy(table_hbm.at[gidx[b]], gath[b], gsem[b])

    zero16 = jnp.zeros((16,), jnp.float32)
    for r in range(16):
        for q in range(UNITS // 16):
            zbuf[r, pl.ds(q * 16, 16)] = zero16
    row0 = sid * ROWS_W

    def _zero(t, carry):
        pltpu.sync_copy(zbuf, acc.at[pl.ds(row0 + t * 16, 16)])
        return carry

    lax.fori_loop(0, ROWS_W // 16, _zero, 0)
    plsc.subcore_barrier()

    def _pipe(i, carry):
        for b in range(NBUF):
            c = i * NBUF + b
            b2 = (b + HALF) % NBUF

            pltpu.make_async_copy(
                table_hbm.at[gidx[b]], gath[b], gsem[b]).wait()
            o = pl.multiple_of(c * CH, CH)

            @plsc.parallel_loop(0, CH // 16, unroll=2)
            def _scale(g, _b=b, _o=o):
                v16 = vals_all[pl.ds(_o + g * 16, 16)]
                for lane in range(16):
                    sp = _lane_splat(v16, lane)
                    j = g * 16 + lane
                    for q in range(UNITS // 16):
                        gath[_b][j, pl.ds(q * 16, 16)] = (
                            gath[_b][j, pl.ds(q * 16, 16)] * sp)
            pltpu.async_copy(gath[b], acc.at[sidx[b]], ssem[b], add=True)

            @pl.when(c < NCH - HALF)
            def _(c=c, b2=b2):
                @pl.when(c >= HALF)
                def _():
                    pltpu.make_async_copy(
                        gath[b2], acc.at[sidx[b2]], ssem[b2]).wait()
                _stage(c + HALF, b2)
                pltpu.async_copy(table_hbm.at[gidx[b2]], gath[b2], gsem[b2])
        return carry

    lax.fori_loop(0, NCH // NBUF, _pipe, 0)

    for b in range(NBUF):
        pltpu.make_async_copy(gath[b], acc.at[sidx[b]], ssem[b]).wait()

    plsc.subcore_barrier()
    pltpu.sync_copy(acc.at[pl.ds(row0, ROWS_W)],
                    out_hbm.at[cid, pl.ds(row0, ROWS_W), pl.ds(0, UNITS)])


def _concat_body(p_ref, d_ref, b_ref, o_ref):
    o_ref[:, :UNITS] = (p_ref[0, :, :UNITS] + p_ref[1, :, :UNITS]
                        + b_ref[...])
    o_ref[:, UNITS:] = d_ref[...]


_RB = 1024


def _concat(partials, dense_feat, bias2d):
    return pl.pallas_call(
        _concat_body,
        grid=(B // _RB,),
        in_specs=[
            pl.BlockSpec((NC, _RB, 2 * UNITS), lambda i: (0, i, 0)),
            pl.BlockSpec((_RB, D_DENSE), lambda i: (i, 0)),
            pl.BlockSpec((1, UNITS), lambda i: (0, 0)),
        ],
        out_specs=pl.BlockSpec((_RB, UNITS + D_DENSE), lambda i: (i, 0)),
        out_shape=jax.ShapeDtypeStruct((B, UNITS + D_DENSE), jnp.float32),
    )(partials, dense_feat, bias2d)


def kernel(sparse_rows, sparse_cols, sparse_values, dense_feat, kernel, bias):
    cols = sparse_cols.astype(jnp.int32)
    rows = sparse_rows.astype(jnp.int32)
    partials = _sc_partial(cols, rows, sparse_values, kernel)
    return _concat(partials, dense_feat, bias.reshape(1, UNITS))
